# Initial kernel scaffold; baseline (speedup 1.0000x reference)
import functools

import jax
import jax.numpy as jnp
from jax import lax
from jax.experimental import pallas as pl
from jax.experimental.pallas import tpu as pltpu
from jax.experimental.pallas import tpu_sc as plsc

B_ = 64
F_ = 8192
K_ = 4
CO_ = 16
ROWS = 256


def _topk_body(d_ref, idx_ref):
    d = d_ref[...]
    iota = lax.broadcasted_iota(jnp.int32, (ROWS, F_), 1)
    big = jnp.int32(2 ** 30)
    inf = jnp.float32(jnp.inf)
    for t in range(K_):
        m = jnp.min(d, axis=1, keepdims=True)
        im = jnp.min(jnp.where(d == m, iota, big), axis=1)
        idx_ref[:, t] = im
        if t < K_ - 1:
            d = jnp.where(iota == im[:, None], inf, d)


def _topk(d2):
    return pl.pallas_call(
        _topk_body,
        grid=(F_ // ROWS,),
        in_specs=[pl.BlockSpec((ROWS, F_), lambda i: (i, 0))],
        out_specs=pl.BlockSpec((ROWS, K_), lambda i: (i, 0)),
        out_shape=jax.ShapeDtypeStruct((F_, K_), jnp.int32),
    )(d2)


def _sc_gather(x2, c2, idx_flat):
    mesh = plsc.VectorSubcoreMesh(core_axis_name="c", subcore_axis_name="s")

    @functools.partial(
        pl.kernel,
        out_type=[jax.ShapeDtypeStruct((B_, K_ * F_), jnp.float32)] * 2,
        mesh=mesh,
        scratch_types=[
            pltpu.VMEM((K_ * F_,), jnp.int32),
            pltpu.VMEM((F_,), jnp.float32),
            pltpu.VMEM((K_ * F_,), jnp.float32),
        ],
    )
    def k(x_hbm, c_hbm, idx_hbm, gx_hbm, gc_hbm, idx_v, row_v, out_v):
        wid = lax.axis_index("s") * 2 + lax.axis_index("c")
        pltpu.sync_copy(idx_hbm, idx_v)
        n_chunks = (K_ * F_) // 16
        for p in range(4):
            pid = p * 32 + wid
            b = pid % B_
            src = x_hbm if p < 2 else c_hbm
            dst = gx_hbm if p < 2 else gc_hbm
            pltpu.sync_copy(src.at[b], row_v)

            def body(j, _):
                off = j * 16
                iv = idx_v[pl.ds(off, 16)]
                out_v[pl.ds(off, 16)] = plsc.load_gather(row_v, [iv])
                return 0

            lax.fori_loop(0, n_chunks, body, 0, unroll=8)
            pltpu.sync_copy(out_v, dst.at[b])

    return k(x2, c2, idx_flat)


def _conv_body(g_ref, w_ref, b_ref, o_ref):
    g = g_ref[0]
    w = w_ref[...]
    bb = b_ref[...]
    y = lax.dot_general(w, g, (((1,), (0,)), ((), ())),
                        preferred_element_type=jnp.float32)
    o_ref[0] = jnp.maximum(y + bb, 0.0)


def _conv(g, w, b2):
    return pl.pallas_call(
        _conv_body,
        grid=(B_,),
        in_specs=[
            pl.BlockSpec((1, K_, F_), lambda i: (i, 0, 0)),
            pl.BlockSpec((CO_, K_), lambda i: (0, 0)),
            pl.BlockSpec((CO_, 1), lambda i: (0, 0)),
        ],
        out_specs=pl.BlockSpec((1, CO_, F_), lambda i: (i, 0, 0)),
        out_shape=jax.ShapeDtypeStruct((B_, CO_, F_), jnp.float32),
    )(g, w, b2)


def kernel(X, Coord, distances, W, b):
    d2 = distances[0]
    idx = _topk(d2)
    idx_flat = idx.T.reshape(-1)
    x2 = X[:, 0, :]
    c2 = Coord[:, 0, :]
    gx, gc = _sc_gather(x2, c2, idx_flat)
    gx = gx.reshape(B_, K_, F_)
    gc = gc.reshape(B_, K_, F_)
    w2 = W[:, 0, :]
    b2 = b.reshape(CO_, 1)
    return (_conv(gx, w2, b2), _conv(gc, w2, b2))


# speedup vs baseline: 41.9579x; 41.9579x over previous
import functools

import jax
import jax.numpy as jnp
from jax import lax
from jax.experimental import pallas as pl
from jax.experimental.pallas import tpu as pltpu
from jax.experimental.pallas import tpu_sc as plsc

B_ = 64
F_ = 8192
K_ = 4
CO_ = 16
ROWS = 256


def _topk_body(d_ref, idx_ref):
    d = d_ref[...]
    iota = lax.broadcasted_iota(jnp.int32, (ROWS, F_), 1)
    big = jnp.int32(2 ** 30)
    inf = jnp.float32(jnp.inf)
    for t in range(K_):
        m = jnp.min(d, axis=1, keepdims=True)
        im = jnp.min(jnp.where(d == m, iota, big), axis=1)
        idx_ref[:, t] = im
        if t < K_ - 1:
            d = jnp.where(iota == im[:, None], inf, d)


def _topk(d2):
    return pl.pallas_call(
        _topk_body,
        grid=(F_ // ROWS,),
        in_specs=[pl.BlockSpec((ROWS, F_), lambda i: (i, 0))],
        out_specs=pl.BlockSpec((ROWS, K_), lambda i: (i, 0)),
        out_shape=jax.ShapeDtypeStruct((F_, K_), jnp.int32),
    )(d2)


def _sc_gather(x2, c2, idx_flat):
    mesh = plsc.VectorSubcoreMesh(core_axis_name="c", subcore_axis_name="s")

    @functools.partial(
        pl.kernel,
        out_type=[jax.ShapeDtypeStruct((B_, K_ * F_), jnp.float32)] * 2,
        mesh=mesh,
        scratch_types=[
            pltpu.VMEM((K_ * F_,), jnp.int32),
            pltpu.VMEM((F_,), jnp.float32),
            pltpu.VMEM((K_ * F_,), jnp.float32),
        ],
        compiler_params=pltpu.CompilerParams(needs_layout_passes=False),
    )
    def k(x_hbm, c_hbm, idx_hbm, gx_hbm, gc_hbm, idx_v, row_v, out_v):
        wid = lax.axis_index("s") * 2 + lax.axis_index("c")
        pltpu.sync_copy(idx_hbm, idx_v)
        n_chunks = (K_ * F_) // 16
        for p in range(4):
            pid = p * 32 + wid
            b = pid % B_
            src = x_hbm if p < 2 else c_hbm
            dst = gx_hbm if p < 2 else gc_hbm
            pltpu.sync_copy(src.at[b], row_v)

            def body(j, _):
                off = j * 16
                iv = idx_v[pl.ds(off, 16)]
                out_v[pl.ds(off, 16)] = plsc.load_gather(row_v, [iv])
                return 0

            lax.fori_loop(0, n_chunks, body, 0, unroll=8)
            pltpu.sync_copy(out_v, dst.at[b])

    return k(x2, c2, idx_flat)


def _conv_body(g_ref, w_ref, b_ref, o_ref):
    g = g_ref[0]
    w = w_ref[...]
    bb = b_ref[...]
    y = lax.dot_general(w, g, (((1,), (0,)), ((), ())),
                        preferred_element_type=jnp.float32)
    o_ref[0] = jnp.maximum(y + bb, 0.0)


def _conv(g, w, b2):
    return pl.pallas_call(
        _conv_body,
        grid=(B_,),
        in_specs=[
            pl.BlockSpec((1, K_, F_), lambda i: (i, 0, 0)),
            pl.BlockSpec((CO_, K_), lambda i: (0, 0)),
            pl.BlockSpec((CO_, 1), lambda i: (0, 0)),
        ],
        out_specs=pl.BlockSpec((1, CO_, F_), lambda i: (i, 0, 0)),
        out_shape=jax.ShapeDtypeStruct((B_, CO_, F_), jnp.float32),
    )(g, w, b2)


def kernel(X, Coord, distances, W, b):
    d2 = distances[0]
    idx = _topk(d2)
    idx_flat = idx.T.reshape(-1)
    x2 = X[:, 0, :]
    c2 = Coord[:, 0, :]
    gx, gc = _sc_gather(x2, c2, idx_flat)
    gx = gx.reshape(B_, K_, F_)
    gc = gc.reshape(B_, K_, F_)
    w2 = W[:, 0, :]
    b2 = b.reshape(CO_, 1)
    return (_conv(gx, w2, b2), _conv(gc, w2, b2))
